# baseline (speedup 1.0000x reference)
import functools

import jax
import jax.numpy as jnp
from jax import lax
from jax.experimental import pallas as pl
from jax.experimental.pallas import tpu as pltpu
from jax.experimental.pallas import tpu_sc as plsc

V = 1000000
D = 64
L = 50
B = 1024
H = 128
NUM_LANG = 6

_NC, _NS = 2, 16
_NW = _NC * _NS
_N_TOK = B * L
_B_PER_W = _N_TOK // _NW

_BB = 128


_CH = 2
_B_PER_CH = _B_PER_W // _CH


def _sc_gather(table2, idx_half):
    mesh = plsc.VectorSubcoreMesh(
        core_axis_name="c", subcore_axis_name="s",
        num_cores=_NC, num_subcores=_NS)

    @functools.partial(
        pl.kernel,
        out_type=jax.ShapeDtypeStruct((_N_TOK, 2 * D), jnp.float32),
        mesh=mesh,
        scratch_types=[
            pltpu.VMEM((_B_PER_W,), jnp.int32),
            pltpu.VMEM((_B_PER_CH, 2 * D), jnp.float32),
            pltpu.SemaphoreType.DMA,
        ],
    )
    def gk(table_hbm, idx_hbm, out_hbm, idx_v, rows_v, sem):
        wid = lax.axis_index("s") * _NC + lax.axis_index("c")
        base = wid * _B_PER_W
        pltpu.sync_copy(idx_hbm.at[pl.ds(base, _B_PER_W)], idx_v)
        for ch in range(_CH):
            pltpu.async_copy(
                table_hbm.at[idx_v.at[pl.ds(ch * _B_PER_CH, _B_PER_CH)]],
                rows_v, sem).wait()
            pltpu.sync_copy(
                rows_v, out_hbm.at[pl.ds(base + ch * _B_PER_CH, _B_PER_CH)])

    return gk(table2, idx_half)


def _lstm_step(x, h, c, wx, wh, b):
    z = (jnp.dot(x, wx, preferred_element_type=jnp.float32)
         + jnp.dot(h, wh, preferred_element_type=jnp.float32) + b)
    i = jax.nn.sigmoid(z[:, 0:H])
    f = jax.nn.sigmoid(z[:, H:2 * H])
    g = jnp.tanh(z[:, 2 * H:3 * H])
    o = jax.nn.sigmoid(z[:, 3 * H:4 * H])
    c2 = f * c + i * g
    h2 = o * jnp.tanh(c2)
    return h2, c2


def _sel_half(ew, p):
    return jnp.where(p != 0, ew[..., D:2 * D], ew[..., 0:D])


def _lstm_kernel(ef_ref, eb_ref, pf_ref, pb_ref, wxf_ref, whf_ref, bf_ref,
                 wxb_ref, whb_ref, bb_ref, hf_out, hb_out,
                 hf_c, cf_c, hb_c, cb_c):
    @pl.when(pl.program_id(0) == 0)
    def _init():
        z = jnp.zeros((B, H), jnp.float32)
        hf_c[...] = z
        cf_c[...] = z
        hb_c[...] = z
        cb_c[...] = z

    xf = _sel_half(ef_ref[0], pf_ref[0])
    h2, c2 = _lstm_step(xf, hf_c[...], cf_c[...],
                        wxf_ref[...], whf_ref[...], bf_ref[...])
    hf_out[0] = h2
    hf_c[...] = h2
    cf_c[...] = c2

    xb = _sel_half(eb_ref[0], pb_ref[0])
    h2, c2 = _lstm_step(xb, hb_c[...], cb_c[...],
                        wxb_ref[...], whb_ref[...], bb_ref[...])
    hb_out[0] = h2
    hb_c[...] = h2
    cb_c[...] = c2


def _lstm_tc(embw_t, par_t, wx_f, wh_f, b_f, wx_b, wh_b, b_b):
    full2 = lambda t: (0, 0)
    full1 = lambda t: (0,)
    return pl.pallas_call(
        _lstm_kernel,
        grid=(L,),
        in_specs=[
            pl.BlockSpec((1, B, 2 * D), lambda t: (t, 0, 0)),
            pl.BlockSpec((1, B, 2 * D), lambda t: (L - 1 - t, 0, 0)),
            pl.BlockSpec((1, B, 1), lambda t: (t, 0, 0)),
            pl.BlockSpec((1, B, 1), lambda t: (L - 1 - t, 0, 0)),
            pl.BlockSpec((D, 4 * H), full2),
            pl.BlockSpec((H, 4 * H), full2),
            pl.BlockSpec((4 * H,), full1),
            pl.BlockSpec((D, 4 * H), full2),
            pl.BlockSpec((H, 4 * H), full2),
            pl.BlockSpec((4 * H,), full1),
        ],
        out_specs=[
            pl.BlockSpec((1, B, H), lambda t: (t, 0, 0)),
            pl.BlockSpec((1, B, H), lambda t: (L - 1 - t, 0, 0)),
        ],
        out_shape=[
            jax.ShapeDtypeStruct((L, B, H), jnp.float32),
            jax.ShapeDtypeStruct((L, B, H), jnp.float32),
        ],
        scratch_shapes=[pltpu.VMEM((B, H), jnp.float32)] * 4,
    )(embw_t, embw_t, par_t, par_t, wx_f, wh_f, b_f, wx_b, wh_b, b_b)


def _heads_kernel(e_ref, p_ref, hf_ref, hb_ref,
                  c1w_ref, c1b_ref, c2w_ref, c2b_ref,
                  l1wf_ref, l1wb_ref, l1b_ref, l2w_ref, l2b_ref,
                  l3w_ref, l3b_ref,
                  cs1wf_ref, cs1wb_ref, cs1b_ref, cs2wt_ref, cs2b_ref,
                  f1wf_ref, f1wb_ref, f1b_ref, f2w_ref, f2b_ref,
                  lang_ref, cs_ref, form_ref, char_ref):
    dot = functools.partial(jnp.dot, preferred_element_type=jnp.float32)

    e = _sel_half(e_ref[...], p_ref[...])
    zp = jnp.zeros((2, _BB, D), jnp.float32)
    ep = jnp.concatenate([zp, e, zp], axis=0)
    acc = None
    for k in range(3):
        xs = ep[1 + k:1 + k + L].reshape(L * _BB, D)
        t = dot(xs, c1w_ref[k])
        acc = t if acc is None else acc + t
    y1 = jax.nn.relu(acc + c1b_ref[...])
    zp1 = jnp.zeros((2, _BB, 128), jnp.float32)
    y1p = jnp.concatenate([zp1, y1.reshape(L, _BB, 128), zp1], axis=0)
    acc = None
    for k in range(5):
        xs = y1p[k:k + L].reshape(L * _BB, 128)
        t = dot(xs, c2w_ref[k])
        acc = t if acc is None else acc + t
    y2 = jax.nn.relu(acc + c2b_ref[...])
    char_ref[...] = jnp.max(y2.reshape(L, _BB, 128), axis=0)

    hf = hf_ref[...]
    hb = hb_ref[...]
    gapf = jnp.mean(hf, axis=0)
    gapb = jnp.mean(hb, axis=0)
    z1 = jax.nn.relu(dot(gapf, l1wf_ref[...]) + dot(gapb, l1wb_ref[...])
                     + l1b_ref[...])
    z2 = jax.nn.relu(dot(z1, l2w_ref[...]) + l2b_ref[...])
    logits = dot(z2, l3w_ref[...]) + l3b_ref[...]
    lang_ref[...] = jax.nn.softmax(logits, axis=-1)

    fz = jax.nn.relu(dot(gapf, f1wf_ref[...]) + dot(gapb, f1wb_ref[...])
                     + f1b_ref[...])
    flog = dot(fz, f2w_ref[...]) + f2b_ref[...]
    form_ref[...] = jax.nn.softmax(flog, axis=-1)

    hff = hf.reshape(L * _BB, H)
    hbf = hb.reshape(L * _BB, H)
    csz = jax.nn.relu(dot(hff, cs1wf_ref[...]) + dot(hbf, cs1wb_ref[...])
                      + cs1b_ref[...])
    csv = jnp.sum(csz * cs2wt_ref[...], axis=1, keepdims=True) + cs2b_ref[...]
    cs_ref[...] = jax.nn.sigmoid(csv).reshape(L, _BB, 1)


def _heads_tc(embw_t, par_t, hf, hb, c1w, c1b, c2w, c2b,
              l1wf, l1wb, l1b, l2w, l2b, l3w, l3b,
              cs1wf, cs1wb, cs1b, cs2wt, cs2b,
              f1wf, f1wb, f1b, f2w, f2b):
    def full(shape):
        n = len(shape)
        return pl.BlockSpec(shape, lambda i, _n=n: (0,) * _n)

    return pl.pallas_call(
        _heads_kernel,
        grid=(B // _BB,),
        in_specs=[
            pl.BlockSpec((L, _BB, 2 * D), lambda i: (0, i, 0)),
            pl.BlockSpec((L, _BB, 1), lambda i: (0, i, 0)),
            pl.BlockSpec((L, _BB, H), lambda i: (0, i, 0)),
            pl.BlockSpec((L, _BB, H), lambda i: (0, i, 0)),
            full((3, D, 128)), full((128,)),
            full((5, 128, 128)), full((128,)),
            full((H, 256)), full((H, 256)), full((256,)),
            full((256, 128)), full((128,)),
            full((128, NUM_LANG)), full((NUM_LANG,)),
            full((H, 64)), full((H, 64)), full((64,)),
            full((1, 64)), full((1,)),
            full((H, 64)), full((H, 64)), full((64,)),
            full((64, 3)), full((3,)),
        ],
        out_specs=[
            pl.BlockSpec((_BB, NUM_LANG), lambda i: (i, 0)),
            pl.BlockSpec((L, _BB, 1), lambda i: (0, i, 0)),
            pl.BlockSpec((_BB, 3), lambda i: (i, 0)),
            pl.BlockSpec((_BB, 128), lambda i: (i, 0)),
        ],
        out_shape=[
            jax.ShapeDtypeStruct((B, NUM_LANG), jnp.float32),
            jax.ShapeDtypeStruct((L, B, 1), jnp.float32),
            jax.ShapeDtypeStruct((B, 3), jnp.float32),
            jax.ShapeDtypeStruct((B, 128), jnp.float32),
        ],
    )(embw_t, par_t, hf, hb, c1w, c1b, c2w, c2b,
      l1wf, l1wb, l1b, l2w, l2b, l3w, l3b,
      cs1wf, cs1wb, cs1b, cs2wt, cs2b,
      f1wf, f1wb, f1b, f2w, f2b)


def kernel(token_ids, table, c1w, c1b, c2w, c2b, wx_f, wh_f, b_f,
           wx_b, wh_b, b_b, l1w, l1b, l2w, l2b, l3w, l3b,
           cs1w, cs1b, cs2w, cs2b, f1w, f1b, f2w, f2b):
    idx_t = token_ids.astype(jnp.int32).T.reshape(-1)
    table2 = table.reshape(V // 2, 2 * D)
    embw_t = _sc_gather(table2, idx_t // 2).reshape(L, B, 2 * D)
    par_t = (idx_t % 2).reshape(L, B, 1)
    hf, hb = _lstm_tc(embw_t, par_t, wx_f, wh_f, b_f, wx_b, wh_b, b_b)
    lang, cs, form, char = _heads_tc(
        embw_t, par_t, hf, hb, c1w, c1b, c2w, c2b,
        l1w[:H], l1w[H:], l1b, l2w, l2b, l3w, l3b,
        cs1w[:H], cs1w[H:], cs1b, cs2w.T, cs2b,
        f1w[:H], f1w[H:], f1b, f2w, f2b)
    code_switch = jnp.transpose(cs, (1, 0, 2))
    return lang, code_switch, form, char


# speedup vs baseline: 1.4883x; 1.4883x over previous
import functools

import jax
import jax.numpy as jnp
from jax import lax
from jax.experimental import pallas as pl
from jax.experimental.pallas import tpu as pltpu
from jax.experimental.pallas import tpu_sc as plsc

V = 1000000
D = 64
L = 50
B = 1024
H = 128
NUM_LANG = 6

_NC, _NS = 2, 16
_NW = _NC * _NS
_N_TOK = B * L
_B_PER_W = _N_TOK // _NW

_BB = 128


_CH = 4
_B_PER_CH = _B_PER_W // _CH


_CBW = 4096


def _widen_kernel(tt_ref, out_ref):
    out_ref[:, 0:D] = tt_ref[...].T


def _widen_tc(table_t):
    return pl.pallas_call(
        _widen_kernel,
        grid=(pl.cdiv(V, _CBW),),
        in_specs=[pl.BlockSpec((D, _CBW), lambda i: (0, i))],
        out_specs=pl.BlockSpec((_CBW, 2 * D), lambda i: (i, 0)),
        out_shape=jax.ShapeDtypeStruct((V, 2 * D), jnp.float32),
    )(table_t)


def _pad_tok_kernel(tok_ref, out_ref):
    out_ref[...] = jnp.zeros((B, 128), jnp.int32)
    out_ref[:, 0:L] = tok_ref[...]


def _pad_tok_tc(token_ids):
    return pl.pallas_call(
        _pad_tok_kernel,
        out_shape=jax.ShapeDtypeStruct((B, 128), jnp.int32),
    )(token_ids)


def _sc_gather(table, tok):
    mesh = plsc.VectorSubcoreMesh(
        core_axis_name="c", subcore_axis_name="s",
        num_cores=_NC, num_subcores=_NS)

    @functools.partial(
        pl.kernel,
        out_type=jax.ShapeDtypeStruct((_N_TOK, 2 * D), jnp.float32),
        mesh=mesh,
        scratch_types=[
            pltpu.VMEM((B, D), jnp.int32),
            pltpu.VMEM((_B_PER_W,), jnp.int32),
            pltpu.VMEM((_B_PER_CH, 2 * D), jnp.float32),
            pltpu.SemaphoreType.DMA,
        ],
        compiler_params=pltpu.CompilerParams(
            use_tc_tiling_on_sc=False, needs_layout_passes=False),
    )
    def gk(table_hbm, tok_hbm, out_hbm, tok_v, idx_v, rows_v, sem):
        wid = lax.axis_index("s") * _NC + lax.axis_index("c")
        base = wid * _B_PER_W
        pltpu.sync_copy(tok_hbm.at[:, pl.ds(0, D)], tok_v)
        for k in range(_B_PER_W // 16):
            m = lax.iota(jnp.int32, 16) + (base + 16 * k)
            bv = jnp.bitwise_and(m, B - 1)
            tv = lax.shift_right_logical(m, 10)
            idx_v[pl.ds(16 * k, 16)] = plsc.load_gather(tok_v, [bv, tv])
        for ch in range(_CH):
            pltpu.async_copy(
                table_hbm.at[idx_v.at[pl.ds(ch * _B_PER_CH, _B_PER_CH)]],
                rows_v, sem).wait()
            pltpu.sync_copy(
                rows_v, out_hbm.at[pl.ds(base + ch * _B_PER_CH, _B_PER_CH)])

    return gk(table, tok)


def _lstm_step(x, h, c, wx, wh, b):
    z = (jnp.dot(x, wx, preferred_element_type=jnp.float32)
         + jnp.dot(h, wh, preferred_element_type=jnp.float32) + b)
    i = jax.nn.sigmoid(z[:, 0:H])
    f = jax.nn.sigmoid(z[:, H:2 * H])
    g = jnp.tanh(z[:, 2 * H:3 * H])
    o = jax.nn.sigmoid(z[:, 3 * H:4 * H])
    c2 = f * c + i * g
    h2 = o * jnp.tanh(c2)
    return h2, c2


def _lstm_kernel(ef_ref, eb_ref, wxf_ref, whf_ref, bf_ref,
                 wxb_ref, whb_ref, bb_ref, hf_out, hb_out,
                 hf_c, cf_c, hb_c, cb_c):
    @pl.when(pl.program_id(0) == 0)
    def _init():
        z = jnp.zeros((B, H), jnp.float32)
        hf_c[...] = z
        cf_c[...] = z
        hb_c[...] = z
        cb_c[...] = z

    h2, c2 = _lstm_step(ef_ref[0][:, 0:D], hf_c[...], cf_c[...],
                        wxf_ref[...], whf_ref[...], bf_ref[...])
    hf_out[0] = h2
    hf_c[...] = h2
    cf_c[...] = c2

    h2, c2 = _lstm_step(eb_ref[0][:, 0:D], hb_c[...], cb_c[...],
                        wxb_ref[...], whb_ref[...], bb_ref[...])
    hb_out[0] = h2
    hb_c[...] = h2
    cb_c[...] = c2


def _lstm_tc(emb_t, wx_f, wh_f, b_f, wx_b, wh_b, b_b):
    full2 = lambda t: (0, 0)
    full1 = lambda t: (0,)
    return pl.pallas_call(
        _lstm_kernel,
        grid=(L,),
        in_specs=[
            pl.BlockSpec((1, B, 2 * D), lambda t: (t, 0, 0)),
            pl.BlockSpec((1, B, 2 * D), lambda t: (L - 1 - t, 0, 0)),
            pl.BlockSpec((D, 4 * H), full2),
            pl.BlockSpec((H, 4 * H), full2),
            pl.BlockSpec((4 * H,), full1),
            pl.BlockSpec((D, 4 * H), full2),
            pl.BlockSpec((H, 4 * H), full2),
            pl.BlockSpec((4 * H,), full1),
        ],
        out_specs=[
            pl.BlockSpec((1, B, H), lambda t: (t, 0, 0)),
            pl.BlockSpec((1, B, H), lambda t: (L - 1 - t, 0, 0)),
        ],
        out_shape=[
            jax.ShapeDtypeStruct((L, B, H), jnp.float32),
            jax.ShapeDtypeStruct((L, B, H), jnp.float32),
        ],
        scratch_shapes=[pltpu.VMEM((B, H), jnp.float32)] * 4,
    )(emb_t, emb_t, wx_f, wh_f, b_f, wx_b, wh_b, b_b)


def _heads_kernel(e_ref, hf_ref, hb_ref,
                  c1w_ref, c1b_ref, c2w_ref, c2b_ref,
                  l1wf_ref, l1wb_ref, l1b_ref, l2w_ref, l2b_ref,
                  l3w_ref, l3b_ref,
                  cs1wf_ref, cs1wb_ref, cs1b_ref, cs2wt_ref, cs2b_ref,
                  f1wf_ref, f1wb_ref, f1b_ref, f2w_ref, f2b_ref,
                  lang_ref, cs_ref, form_ref, char_ref):
    dot = functools.partial(jnp.dot, preferred_element_type=jnp.float32)

    e = e_ref[..., 0:D]
    zp = jnp.zeros((2, _BB, D), jnp.float32)
    ep = jnp.concatenate([zp, e, zp], axis=0)
    acc = None
    for k in range(3):
        xs = ep[1 + k:1 + k + L].reshape(L * _BB, D)
        t = dot(xs, c1w_ref[k])
        acc = t if acc is None else acc + t
    y1 = jax.nn.relu(acc + c1b_ref[...])
    zp1 = jnp.zeros((2, _BB, 128), jnp.float32)
    y1p = jnp.concatenate([zp1, y1.reshape(L, _BB, 128), zp1], axis=0)
    acc = None
    for k in range(5):
        xs = y1p[k:k + L].reshape(L * _BB, 128)
        t = dot(xs, c2w_ref[k])
        acc = t if acc is None else acc + t
    y2 = jax.nn.relu(acc + c2b_ref[...])
    char_ref[...] = jnp.max(y2.reshape(L, _BB, 128), axis=0)

    hf = hf_ref[...]
    hb = hb_ref[...]
    gapf = jnp.mean(hf, axis=0)
    gapb = jnp.mean(hb, axis=0)
    z1 = jax.nn.relu(dot(gapf, l1wf_ref[...]) + dot(gapb, l1wb_ref[...])
                     + l1b_ref[...])
    z2 = jax.nn.relu(dot(z1, l2w_ref[...]) + l2b_ref[...])
    logits = dot(z2, l3w_ref[...]) + l3b_ref[...]
    lang_ref[...] = jax.nn.softmax(logits, axis=-1)

    fz = jax.nn.relu(dot(gapf, f1wf_ref[...]) + dot(gapb, f1wb_ref[...])
                     + f1b_ref[...])
    flog = dot(fz, f2w_ref[...]) + f2b_ref[...]
    form_ref[...] = jax.nn.softmax(flog, axis=-1)

    hff = hf.reshape(L * _BB, H)
    hbf = hb.reshape(L * _BB, H)
    csz = jax.nn.relu(dot(hff, cs1wf_ref[...]) + dot(hbf, cs1wb_ref[...])
                      + cs1b_ref[...])
    csv = jnp.sum(csz * cs2wt_ref[...], axis=1, keepdims=True) + cs2b_ref[...]
    cs_ref[...] = jax.nn.sigmoid(csv).reshape(L, _BB, 1)


def _heads_tc(emb_t, hf, hb, c1w, c1b, c2w, c2b,
              l1wf, l1wb, l1b, l2w, l2b, l3w, l3b,
              cs1wf, cs1wb, cs1b, cs2wt, cs2b,
              f1wf, f1wb, f1b, f2w, f2b):
    def full(shape):
        n = len(shape)
        return pl.BlockSpec(shape, lambda i, _n=n: (0,) * _n)

    return pl.pallas_call(
        _heads_kernel,
        grid=(B // _BB,),
        in_specs=[
            pl.BlockSpec((L, _BB, 2 * D), lambda i: (0, i, 0)),
            pl.BlockSpec((L, _BB, H), lambda i: (0, i, 0)),
            pl.BlockSpec((L, _BB, H), lambda i: (0, i, 0)),
            full((3, D, 128)), full((128,)),
            full((5, 128, 128)), full((128,)),
            full((H, 256)), full((H, 256)), full((256,)),
            full((256, 128)), full((128,)),
            full((128, NUM_LANG)), full((NUM_LANG,)),
            full((H, 64)), full((H, 64)), full((64,)),
            full((1, 64)), full((1,)),
            full((H, 64)), full((H, 64)), full((64,)),
            full((64, 3)), full((3,)),
        ],
        out_specs=[
            pl.BlockSpec((_BB, NUM_LANG), lambda i: (i, 0)),
            pl.BlockSpec((L, _BB, 1), lambda i: (0, i, 0)),
            pl.BlockSpec((_BB, 3), lambda i: (i, 0)),
            pl.BlockSpec((_BB, 128), lambda i: (i, 0)),
        ],
        out_shape=[
            jax.ShapeDtypeStruct((B, NUM_LANG), jnp.float32),
            jax.ShapeDtypeStruct((L, B, 1), jnp.float32),
            jax.ShapeDtypeStruct((B, 3), jnp.float32),
            jax.ShapeDtypeStruct((B, 128), jnp.float32),
        ],
    )(emb_t, hf, hb, c1w, c1b, c2w, c2b,
      l1wf, l1wb, l1b, l2w, l2b, l3w, l3b,
      cs1wf, cs1wb, cs1b, cs2wt, cs2b,
      f1wf, f1wb, f1b, f2w, f2b)


def kernel(token_ids, table, c1w, c1b, c2w, c2b, wx_f, wh_f, b_f,
           wx_b, wh_b, b_b, l1w, l1b, l2w, l2b, l3w, l3b,
           cs1w, cs1b, cs2w, cs2b, f1w, f1b, f2w, f2b):
    tok128 = _pad_tok_tc(token_ids.astype(jnp.int32))
    table_w = _widen_tc(table.T)
    emb_t = _sc_gather(table_w, tok128).reshape(L, B, 2 * D)
    hf, hb = _lstm_tc(emb_t, wx_f, wh_f, b_f, wx_b, wh_b, b_b)
    lang, cs, form, char = _heads_tc(
        emb_t, hf, hb, c1w, c1b, c2w, c2b,
        l1w[:H], l1w[H:], l1b, l2w, l2b, l3w, l3b,
        cs1w[:H], cs1w[H:], cs1b, cs2w.T, cs2b,
        f1w[:H], f1w[H:], f1b, f2w, f2b)
    code_switch = jnp.transpose(cs, (1, 0, 2))
    return lang, code_switch, form, char


# speedup vs baseline: 1.7917x; 1.2038x over previous
import functools

import jax
import jax.numpy as jnp
from jax import lax
from jax.experimental import pallas as pl
from jax.experimental.pallas import tpu as pltpu
from jax.experimental.pallas import tpu_sc as plsc

V = 1000000
D = 64
L = 50
B = 1024
H = 128
NUM_LANG = 6

_NC, _NS = 2, 16
_NW = _NC * _NS
_N_TOK = B * L
_B_PER_W = _N_TOK // _NW

_BB = 128


_CH = 4
_B_PER_CH = _B_PER_W // _CH


_CBW = 16384


def _widen_kernel(tt_ref, out_ref):
    out_ref[:, 0:D] = tt_ref[...].T


def _widen_tc(table_t):
    return pl.pallas_call(
        _widen_kernel,
        grid=(pl.cdiv(V, _CBW),),
        in_specs=[pl.BlockSpec((D, _CBW), lambda i: (0, i))],
        out_specs=pl.BlockSpec((_CBW, 2 * D), lambda i: (i, 0)),
        out_shape=jax.ShapeDtypeStruct((V, 2 * D), jnp.float32),
    )(table_t)


def _pad_tok_kernel(tok_ref, out_ref):
    out_ref[...] = jnp.zeros((B, 128), jnp.int32)
    out_ref[:, 0:L] = tok_ref[...]


def _pad_tok_tc(token_ids):
    return pl.pallas_call(
        _pad_tok_kernel,
        out_shape=jax.ShapeDtypeStruct((B, 128), jnp.int32),
    )(token_ids)


def _sc_gather(table, tok):
    mesh = plsc.VectorSubcoreMesh(
        core_axis_name="c", subcore_axis_name="s",
        num_cores=_NC, num_subcores=_NS)

    @functools.partial(
        pl.kernel,
        out_type=jax.ShapeDtypeStruct((_N_TOK, 2 * D), jnp.float32),
        mesh=mesh,
        scratch_types=[
            pltpu.VMEM((B, D), jnp.int32),
            pltpu.VMEM((_B_PER_W,), jnp.int32),
            pltpu.VMEM((_B_PER_CH, 2 * D), jnp.float32),
            pltpu.SemaphoreType.DMA,
        ],
        compiler_params=pltpu.CompilerParams(
            use_tc_tiling_on_sc=False, needs_layout_passes=False),
    )
    def gk(table_hbm, tok_hbm, out_hbm, tok_v, idx_v, rows_v, sem):
        wid = lax.axis_index("s") * _NC + lax.axis_index("c")
        base = wid * _B_PER_W
        pltpu.sync_copy(tok_hbm.at[:, pl.ds(0, D)], tok_v)
        for k in range(_B_PER_W // 16):
            m = lax.iota(jnp.int32, 16) + (base + 16 * k)
            bv = jnp.bitwise_and(m, B - 1)
            tv = lax.shift_right_logical(m, 10)
            idx_v[pl.ds(16 * k, 16)] = plsc.load_gather(tok_v, [bv, tv])
        for ch in range(_CH):
            pltpu.async_copy(
                table_hbm.at[idx_v.at[pl.ds(ch * _B_PER_CH, _B_PER_CH)]],
                rows_v, sem).wait()
            pltpu.sync_copy(
                rows_v, out_hbm.at[pl.ds(base + ch * _B_PER_CH, _B_PER_CH)])

    return gk(table, tok)


def _lstm_step(x, h, c, wx, wh, b):
    z = (jnp.dot(x, wx, preferred_element_type=jnp.float32)
         + jnp.dot(h, wh, preferred_element_type=jnp.float32) + b)
    i = jax.nn.sigmoid(z[:, 0:H])
    f = jax.nn.sigmoid(z[:, H:2 * H])
    g = jnp.tanh(z[:, 2 * H:3 * H])
    o = jax.nn.sigmoid(z[:, 3 * H:4 * H])
    c2 = f * c + i * g
    h2 = o * jnp.tanh(c2)
    return h2, c2


def _lstm_kernel(ef_ref, eb_ref, wxf_ref, whf_ref, bf_ref,
                 wxb_ref, whb_ref, bb_ref, hf_out, hb_out,
                 hf_c, cf_c, hb_c, cb_c):
    @pl.when(pl.program_id(0) == 0)
    def _init():
        z = jnp.zeros((B, H), jnp.float32)
        hf_c[...] = z
        cf_c[...] = z
        hb_c[...] = z
        cb_c[...] = z

    h2, c2 = _lstm_step(ef_ref[0][:, 0:D], hf_c[...], cf_c[...],
                        wxf_ref[...], whf_ref[...], bf_ref[...])
    hf_out[0] = h2
    hf_c[...] = h2
    cf_c[...] = c2

    h2, c2 = _lstm_step(eb_ref[0][:, 0:D], hb_c[...], cb_c[...],
                        wxb_ref[...], whb_ref[...], bb_ref[...])
    hb_out[0] = h2
    hb_c[...] = h2
    cb_c[...] = c2


def _lstm_tc(emb_t, wx_f, wh_f, b_f, wx_b, wh_b, b_b):
    full2 = lambda t: (0, 0)
    full1 = lambda t: (0,)
    return pl.pallas_call(
        _lstm_kernel,
        grid=(L,),
        in_specs=[
            pl.BlockSpec((1, B, 2 * D), lambda t: (t, 0, 0)),
            pl.BlockSpec((1, B, 2 * D), lambda t: (L - 1 - t, 0, 0)),
            pl.BlockSpec((D, 4 * H), full2),
            pl.BlockSpec((H, 4 * H), full2),
            pl.BlockSpec((4 * H,), full1),
            pl.BlockSpec((D, 4 * H), full2),
            pl.BlockSpec((H, 4 * H), full2),
            pl.BlockSpec((4 * H,), full1),
        ],
        out_specs=[
            pl.BlockSpec((1, B, H), lambda t: (t, 0, 0)),
            pl.BlockSpec((1, B, H), lambda t: (L - 1 - t, 0, 0)),
        ],
        out_shape=[
            jax.ShapeDtypeStruct((L, B, H), jnp.float32),
            jax.ShapeDtypeStruct((L, B, H), jnp.float32),
        ],
        scratch_shapes=[pltpu.VMEM((B, H), jnp.float32)] * 4,
    )(emb_t, emb_t, wx_f, wh_f, b_f, wx_b, wh_b, b_b)


def _heads_kernel(e_ref, hf_ref, hb_ref,
                  c1w_ref, c1b_ref, c2w_ref, c2b_ref,
                  l1wf_ref, l1wb_ref, l1b_ref, l2w_ref, l2b_ref,
                  l3w_ref, l3b_ref,
                  cs1wf_ref, cs1wb_ref, cs1b_ref, cs2wt_ref, cs2b_ref,
                  f1wf_ref, f1wb_ref, f1b_ref, f2w_ref, f2b_ref,
                  lang_ref, cs_ref, form_ref, char_ref):
    dot = functools.partial(jnp.dot, preferred_element_type=jnp.float32)

    e = e_ref[..., 0:D]
    zp = jnp.zeros((2, _BB, D), jnp.float32)
    ep = jnp.concatenate([zp, e, zp], axis=0)
    acc = None
    for k in range(3):
        xs = ep[1 + k:1 + k + L].reshape(L * _BB, D)
        t = dot(xs, c1w_ref[k])
        acc = t if acc is None else acc + t
    y1 = jax.nn.relu(acc + c1b_ref[...])
    zp1 = jnp.zeros((2, _BB, 128), jnp.float32)
    y1p = jnp.concatenate([zp1, y1.reshape(L, _BB, 128), zp1], axis=0)
    acc = None
    for k in range(5):
        xs = y1p[k:k + L].reshape(L * _BB, 128)
        t = dot(xs, c2w_ref[k])
        acc = t if acc is None else acc + t
    y2 = jax.nn.relu(acc + c2b_ref[...])
    char_ref[...] = jnp.max(y2.reshape(L, _BB, 128), axis=0)

    hf = hf_ref[...]
    hb = hb_ref[...]
    gapf = jnp.mean(hf, axis=0)
    gapb = jnp.mean(hb, axis=0)
    z1 = jax.nn.relu(dot(gapf, l1wf_ref[...]) + dot(gapb, l1wb_ref[...])
                     + l1b_ref[...])
    z2 = jax.nn.relu(dot(z1, l2w_ref[...]) + l2b_ref[...])
    logits = dot(z2, l3w_ref[...]) + l3b_ref[...]
    lang_ref[...] = jax.nn.softmax(logits, axis=-1)

    fz = jax.nn.relu(dot(gapf, f1wf_ref[...]) + dot(gapb, f1wb_ref[...])
                     + f1b_ref[...])
    flog = dot(fz, f2w_ref[...]) + f2b_ref[...]
    form_ref[...] = jax.nn.softmax(flog, axis=-1)

    hff = hf.reshape(L * _BB, H)
    hbf = hb.reshape(L * _BB, H)
    csz = jax.nn.relu(dot(hff, cs1wf_ref[...]) + dot(hbf, cs1wb_ref[...])
                      + cs1b_ref[...])
    csv = jnp.sum(csz * cs2wt_ref[...], axis=1, keepdims=True) + cs2b_ref[...]
    cs_ref[...] = jax.nn.sigmoid(csv).reshape(L, _BB, 1)


def _heads_tc(emb_t, hf, hb, c1w, c1b, c2w, c2b,
              l1wf, l1wb, l1b, l2w, l2b, l3w, l3b,
              cs1wf, cs1wb, cs1b, cs2wt, cs2b,
              f1wf, f1wb, f1b, f2w, f2b):
    def full(shape):
        n = len(shape)
        return pl.BlockSpec(shape, lambda i, _n=n: (0,) * _n)

    return pl.pallas_call(
        _heads_kernel,
        grid=(B // _BB,),
        in_specs=[
            pl.BlockSpec((L, _BB, 2 * D), lambda i: (0, i, 0)),
            pl.BlockSpec((L, _BB, H), lambda i: (0, i, 0)),
            pl.BlockSpec((L, _BB, H), lambda i: (0, i, 0)),
            full((3, D, 128)), full((128,)),
            full((5, 128, 128)), full((128,)),
            full((H, 256)), full((H, 256)), full((256,)),
            full((256, 128)), full((128,)),
            full((128, NUM_LANG)), full((NUM_LANG,)),
            full((H, 64)), full((H, 64)), full((64,)),
            full((1, 64)), full((1,)),
            full((H, 64)), full((H, 64)), full((64,)),
            full((64, 3)), full((3,)),
        ],
        out_specs=[
            pl.BlockSpec((_BB, NUM_LANG), lambda i: (i, 0)),
            pl.BlockSpec((L, _BB, 1), lambda i: (0, i, 0)),
            pl.BlockSpec((_BB, 3), lambda i: (i, 0)),
            pl.BlockSpec((_BB, 128), lambda i: (i, 0)),
        ],
        out_shape=[
            jax.ShapeDtypeStruct((B, NUM_LANG), jnp.float32),
            jax.ShapeDtypeStruct((L, B, 1), jnp.float32),
            jax.ShapeDtypeStruct((B, 3), jnp.float32),
            jax.ShapeDtypeStruct((B, 128), jnp.float32),
        ],
    )(emb_t, hf, hb, c1w, c1b, c2w, c2b,
      l1wf, l1wb, l1b, l2w, l2b, l3w, l3b,
      cs1wf, cs1wb, cs1b, cs2wt, cs2b,
      f1wf, f1wb, f1b, f2w, f2b)


def kernel(token_ids, table, c1w, c1b, c2w, c2b, wx_f, wh_f, b_f,
           wx_b, wh_b, b_b, l1w, l1b, l2w, l2b, l3w, l3b,
           cs1w, cs1b, cs2w, cs2b, f1w, f1b, f2w, f2b):
    tok128 = _pad_tok_tc(token_ids.astype(jnp.int32))
    table_w = _widen_tc(table.T)
    emb_t = _sc_gather(table_w, tok128).reshape(L, B, 2 * D)
    hf, hb = _lstm_tc(emb_t, wx_f, wh_f, b_f, wx_b, wh_b, b_b)
    lang, cs, form, char = _heads_tc(
        emb_t, hf, hb, c1w, c1b, c2w, c2b,
        l1w[:H], l1w[H:], l1b, l2w, l2b, l3w, l3b,
        cs1w[:H], cs1w[H:], cs1b, cs2w.T, cs2b,
        f1w[:H], f1w[H:], f1b, f2w, f2b)
    code_switch = jnp.transpose(cs, (1, 0, 2))
    return lang, code_switch, form, char


# speedup vs baseline: 1.8163x; 1.0137x over previous
import functools

import jax
import jax.numpy as jnp
from jax import lax
from jax.experimental import pallas as pl
from jax.experimental.pallas import tpu as pltpu
from jax.experimental.pallas import tpu_sc as plsc

V = 1000000
D = 64
L = 50
B = 1024
H = 128
NUM_LANG = 6

_NC, _NS = 2, 16
_NW = _NC * _NS
_N_TOK = B * L
_B_PER_W = _N_TOK // _NW

_BB = 128


_CH = 4
_B_PER_CH = _B_PER_W // _CH


_CBW = 32768


def _widen_kernel(tt_ref, out_ref):
    out_ref[:, 0:D] = tt_ref[...].T


def _widen_tc(table_t):
    return pl.pallas_call(
        _widen_kernel,
        grid=(pl.cdiv(V, _CBW),),
        in_specs=[pl.BlockSpec((D, _CBW), lambda i: (0, i))],
        out_specs=pl.BlockSpec((_CBW, 2 * D), lambda i: (i, 0)),
        out_shape=jax.ShapeDtypeStruct((V, 2 * D), jnp.float32),
    )(table_t)


def _pad_tok_kernel(tok_ref, out_ref):
    out_ref[...] = jnp.zeros((B, 128), jnp.int32)
    out_ref[:, 0:L] = tok_ref[...]


def _pad_tok_tc(token_ids):
    return pl.pallas_call(
        _pad_tok_kernel,
        out_shape=jax.ShapeDtypeStruct((B, 128), jnp.int32),
    )(token_ids)


def _sc_gather(table, tok):
    mesh = plsc.VectorSubcoreMesh(
        core_axis_name="c", subcore_axis_name="s",
        num_cores=_NC, num_subcores=_NS)

    @functools.partial(
        pl.kernel,
        out_type=jax.ShapeDtypeStruct((_N_TOK, 2 * D), jnp.float32),
        mesh=mesh,
        scratch_types=[
            pltpu.VMEM((B, D), jnp.int32),
            pltpu.VMEM((_B_PER_W,), jnp.int32),
            pltpu.VMEM((_B_PER_CH, 2 * D), jnp.float32),
            pltpu.SemaphoreType.DMA,
        ],
        compiler_params=pltpu.CompilerParams(
            use_tc_tiling_on_sc=False, needs_layout_passes=False),
    )
    def gk(table_hbm, tok_hbm, out_hbm, tok_v, idx_v, rows_v, sem):
        wid = lax.axis_index("s") * _NC + lax.axis_index("c")
        base = wid * _B_PER_W
        pltpu.sync_copy(tok_hbm.at[:, pl.ds(0, D)], tok_v)
        for k in range(_B_PER_W // 16):
            m = lax.iota(jnp.int32, 16) + (base + 16 * k)
            bv = jnp.bitwise_and(m, B - 1)
            tv = lax.shift_right_logical(m, 10)
            idx_v[pl.ds(16 * k, 16)] = plsc.load_gather(tok_v, [bv, tv])
        for ch in range(_CH):
            pltpu.async_copy(
                table_hbm.at[idx_v.at[pl.ds(ch * _B_PER_CH, _B_PER_CH)]],
                rows_v, sem).wait()
            pltpu.sync_copy(
                rows_v, out_hbm.at[pl.ds(base + ch * _B_PER_CH, _B_PER_CH)])

    return gk(table, tok)


def _lstm_step(x, h, c, wx, wh, b):
    z = (jnp.dot(x, wx, preferred_element_type=jnp.float32)
         + jnp.dot(h, wh, preferred_element_type=jnp.float32) + b)
    i = jax.nn.sigmoid(z[:, 0:H])
    f = jax.nn.sigmoid(z[:, H:2 * H])
    g = jnp.tanh(z[:, 2 * H:3 * H])
    o = jax.nn.sigmoid(z[:, 3 * H:4 * H])
    c2 = f * c + i * g
    h2 = o * jnp.tanh(c2)
    return h2, c2


def _lstm_kernel(ef_ref, eb_ref, wxf_ref, whf_ref, bf_ref,
                 wxb_ref, whb_ref, bb_ref, hf_out, hb_out,
                 hf_c, cf_c, hb_c, cb_c):
    @pl.when(pl.program_id(0) == 0)
    def _init():
        z = jnp.zeros((B, H), jnp.float32)
        hf_c[...] = z
        cf_c[...] = z
        hb_c[...] = z
        cb_c[...] = z

    h2, c2 = _lstm_step(ef_ref[0][:, 0:D], hf_c[...], cf_c[...],
                        wxf_ref[...], whf_ref[...], bf_ref[...])
    hf_out[0] = h2
    hf_c[...] = h2
    cf_c[...] = c2

    h2, c2 = _lstm_step(eb_ref[0][:, 0:D], hb_c[...], cb_c[...],
                        wxb_ref[...], whb_ref[...], bb_ref[...])
    hb_out[0] = h2
    hb_c[...] = h2
    cb_c[...] = c2


def _lstm_tc(emb_t, wx_f, wh_f, b_f, wx_b, wh_b, b_b):
    full2 = lambda t: (0, 0)
    full1 = lambda t: (0,)
    return pl.pallas_call(
        _lstm_kernel,
        grid=(L,),
        in_specs=[
            pl.BlockSpec((1, B, 2 * D), lambda t: (t, 0, 0)),
            pl.BlockSpec((1, B, 2 * D), lambda t: (L - 1 - t, 0, 0)),
            pl.BlockSpec((D, 4 * H), full2),
            pl.BlockSpec((H, 4 * H), full2),
            pl.BlockSpec((4 * H,), full1),
            pl.BlockSpec((D, 4 * H), full2),
            pl.BlockSpec((H, 4 * H), full2),
            pl.BlockSpec((4 * H,), full1),
        ],
        out_specs=[
            pl.BlockSpec((1, B, H), lambda t: (t, 0, 0)),
            pl.BlockSpec((1, B, H), lambda t: (L - 1 - t, 0, 0)),
        ],
        out_shape=[
            jax.ShapeDtypeStruct((L, B, H), jnp.float32),
            jax.ShapeDtypeStruct((L, B, H), jnp.float32),
        ],
        scratch_shapes=[pltpu.VMEM((B, H), jnp.float32)] * 4,
    )(emb_t, emb_t, wx_f, wh_f, b_f, wx_b, wh_b, b_b)


def _heads_kernel(e_ref, hf_ref, hb_ref,
                  c1w_ref, c1b_ref, c2w_ref, c2b_ref,
                  l1wf_ref, l1wb_ref, l1b_ref, l2w_ref, l2b_ref,
                  l3w_ref, l3b_ref,
                  cs1wf_ref, cs1wb_ref, cs1b_ref, cs2wt_ref, cs2b_ref,
                  f1wf_ref, f1wb_ref, f1b_ref, f2w_ref, f2b_ref,
                  lang_ref, cs_ref, form_ref, char_ref):
    dot = functools.partial(jnp.dot, preferred_element_type=jnp.float32)

    e = e_ref[..., 0:D]
    zp = jnp.zeros((2, _BB, D), jnp.float32)
    ep = jnp.concatenate([zp, e, zp], axis=0)
    acc = None
    for k in range(3):
        xs = ep[1 + k:1 + k + L].reshape(L * _BB, D)
        t = dot(xs, c1w_ref[k])
        acc = t if acc is None else acc + t
    y1 = jax.nn.relu(acc + c1b_ref[...])
    zp1 = jnp.zeros((2, _BB, 128), jnp.float32)
    y1p = jnp.concatenate([zp1, y1.reshape(L, _BB, 128), zp1], axis=0)
    acc = None
    for k in range(5):
        xs = y1p[k:k + L].reshape(L * _BB, 128)
        t = dot(xs, c2w_ref[k])
        acc = t if acc is None else acc + t
    y2 = jax.nn.relu(acc + c2b_ref[...])
    char_ref[...] = jnp.max(y2.reshape(L, _BB, 128), axis=0)

    hf = hf_ref[...]
    hb = hb_ref[...]
    gapf = jnp.mean(hf, axis=0)
    gapb = jnp.mean(hb, axis=0)
    z1 = jax.nn.relu(dot(gapf, l1wf_ref[...]) + dot(gapb, l1wb_ref[...])
                     + l1b_ref[...])
    z2 = jax.nn.relu(dot(z1, l2w_ref[...]) + l2b_ref[...])
    logits = dot(z2, l3w_ref[...]) + l3b_ref[...]
    lang_ref[...] = jax.nn.softmax(logits, axis=-1)

    fz = jax.nn.relu(dot(gapf, f1wf_ref[...]) + dot(gapb, f1wb_ref[...])
                     + f1b_ref[...])
    flog = dot(fz, f2w_ref[...]) + f2b_ref[...]
    form_ref[...] = jax.nn.softmax(flog, axis=-1)

    hff = hf.reshape(L * _BB, H)
    hbf = hb.reshape(L * _BB, H)
    csz = jax.nn.relu(dot(hff, cs1wf_ref[...]) + dot(hbf, cs1wb_ref[...])
                      + cs1b_ref[...])
    csv = jnp.sum(csz * cs2wt_ref[...], axis=1, keepdims=True) + cs2b_ref[...]
    cs_ref[...] = jax.nn.sigmoid(csv).reshape(L, _BB, 1)


def _heads_tc(emb_t, hf, hb, c1w, c1b, c2w, c2b,
              l1wf, l1wb, l1b, l2w, l2b, l3w, l3b,
              cs1wf, cs1wb, cs1b, cs2wt, cs2b,
              f1wf, f1wb, f1b, f2w, f2b):
    def full(shape):
        n = len(shape)
        return pl.BlockSpec(shape, lambda i, _n=n: (0,) * _n)

    return pl.pallas_call(
        _heads_kernel,
        grid=(B // _BB,),
        in_specs=[
            pl.BlockSpec((L, _BB, 2 * D), lambda i: (0, i, 0)),
            pl.BlockSpec((L, _BB, H), lambda i: (0, i, 0)),
            pl.BlockSpec((L, _BB, H), lambda i: (0, i, 0)),
            full((3, D, 128)), full((128,)),
            full((5, 128, 128)), full((128,)),
            full((H, 256)), full((H, 256)), full((256,)),
            full((256, 128)), full((128,)),
            full((128, NUM_LANG)), full((NUM_LANG,)),
            full((H, 64)), full((H, 64)), full((64,)),
            full((1, 64)), full((1,)),
            full((H, 64)), full((H, 64)), full((64,)),
            full((64, 3)), full((3,)),
        ],
        out_specs=[
            pl.BlockSpec((_BB, NUM_LANG), lambda i: (i, 0)),
            pl.BlockSpec((L, _BB, 1), lambda i: (0, i, 0)),
            pl.BlockSpec((_BB, 3), lambda i: (i, 0)),
            pl.BlockSpec((_BB, 128), lambda i: (i, 0)),
        ],
        out_shape=[
            jax.ShapeDtypeStruct((B, NUM_LANG), jnp.float32),
            jax.ShapeDtypeStruct((L, B, 1), jnp.float32),
            jax.ShapeDtypeStruct((B, 3), jnp.float32),
            jax.ShapeDtypeStruct((B, 128), jnp.float32),
        ],
    )(emb_t, hf, hb, c1w, c1b, c2w, c2b,
      l1wf, l1wb, l1b, l2w, l2b, l3w, l3b,
      cs1wf, cs1wb, cs1b, cs2wt, cs2b,
      f1wf, f1wb, f1b, f2w, f2b)


def kernel(token_ids, table, c1w, c1b, c2w, c2b, wx_f, wh_f, b_f,
           wx_b, wh_b, b_b, l1w, l1b, l2w, l2b, l3w, l3b,
           cs1w, cs1b, cs2w, cs2b, f1w, f1b, f2w, f2b):
    tok128 = _pad_tok_tc(token_ids.astype(jnp.int32))
    table_w = _widen_tc(table.T)
    emb_t = _sc_gather(table_w, tok128).reshape(L, B, 2 * D)
    hf, hb = _lstm_tc(emb_t, wx_f, wh_f, b_f, wx_b, wh_b, b_b)
    lang, cs, form, char = _heads_tc(
        emb_t, hf, hb, c1w, c1b, c2w, c2b,
        l1w[:H], l1w[H:], l1b, l2w, l2b, l3w, l3b,
        cs1w[:H], cs1w[H:], cs1b, cs2w.T, cs2b,
        f1w[:H], f1w[H:], f1b, f2w, f2b)
    code_switch = jnp.transpose(cs, (1, 0, 2))
    return lang, code_switch, form, char


# speedup vs baseline: 1.8618x; 1.0251x over previous
import functools

import jax
import jax.numpy as jnp
from jax import lax
from jax.experimental import pallas as pl
from jax.experimental.pallas import tpu as pltpu
from jax.experimental.pallas import tpu_sc as plsc

V = 1000000
D = 64
L = 50
B = 1024
H = 128
NUM_LANG = 6

_NC, _NS = 2, 16
_NW = _NC * _NS
_N_TOK = B * L
_B_PER_W = _N_TOK // _NW

_BB = 128


_CH = 4
_B_PER_CH = _B_PER_W // _CH


_CBW = 32768


def _widen_kernel(tt_ref, out_ref):
    xt = tt_ref[...].T
    out_ref[...] = jnp.concatenate(
        [xt, jnp.zeros(xt.shape, jnp.float32)], axis=1)


def _widen_tc(table_t):
    return pl.pallas_call(
        _widen_kernel,
        grid=(pl.cdiv(V, _CBW),),
        in_specs=[pl.BlockSpec((D, _CBW), lambda i: (0, i))],
        out_specs=pl.BlockSpec((_CBW, 2 * D), lambda i: (i, 0)),
        out_shape=jax.ShapeDtypeStruct((V, 2 * D), jnp.float32),
    )(table_t)


def _pad_tok_kernel(tok_ref, out_ref):
    out_ref[...] = jnp.zeros((B, 128), jnp.int32)
    out_ref[:, 0:L] = tok_ref[...]


def _pad_tok_tc(token_ids):
    return pl.pallas_call(
        _pad_tok_kernel,
        out_shape=jax.ShapeDtypeStruct((B, 128), jnp.int32),
    )(token_ids)


def _sc_gather(table, tok):
    mesh = plsc.VectorSubcoreMesh(
        core_axis_name="c", subcore_axis_name="s",
        num_cores=_NC, num_subcores=_NS)

    @functools.partial(
        pl.kernel,
        out_type=jax.ShapeDtypeStruct((_N_TOK, 2 * D), jnp.float32),
        mesh=mesh,
        scratch_types=[
            pltpu.VMEM((B, D), jnp.int32),
            pltpu.VMEM((_B_PER_W,), jnp.int32),
            pltpu.VMEM((_B_PER_CH, 2 * D), jnp.float32),
            pltpu.SemaphoreType.DMA,
        ],
        compiler_params=pltpu.CompilerParams(
            use_tc_tiling_on_sc=False, needs_layout_passes=False),
    )
    def gk(table_hbm, tok_hbm, out_hbm, tok_v, idx_v, rows_v, sem):
        wid = lax.axis_index("s") * _NC + lax.axis_index("c")
        base = wid * _B_PER_W
        pltpu.sync_copy(tok_hbm.at[:, pl.ds(0, D)], tok_v)
        for k in range(_B_PER_W // 16):
            m = lax.iota(jnp.int32, 16) + (base + 16 * k)
            bv = jnp.bitwise_and(m, B - 1)
            tv = lax.shift_right_logical(m, 10)
            idx_v[pl.ds(16 * k, 16)] = plsc.load_gather(tok_v, [bv, tv])
        for ch in range(_CH):
            pltpu.async_copy(
                table_hbm.at[idx_v.at[pl.ds(ch * _B_PER_CH, _B_PER_CH)]],
                rows_v, sem).wait()
            pltpu.sync_copy(
                rows_v, out_hbm.at[pl.ds(base + ch * _B_PER_CH, _B_PER_CH)])

    return gk(table, tok)


def _lstm_step(xw, h, c, wxh, b):
    xh = jnp.concatenate([xw, h], axis=1)
    z = jnp.dot(xh, wxh, preferred_element_type=jnp.float32) + b
    i = jax.nn.sigmoid(z[:, 0:H])
    f = jax.nn.sigmoid(z[:, H:2 * H])
    g = jnp.tanh(z[:, 2 * H:3 * H])
    o = jax.nn.sigmoid(z[:, 3 * H:4 * H])
    c2 = f * c + i * g
    h2 = o * jnp.tanh(c2)
    return h2, c2


def _lstm_kernel(ef_ref, eb_ref, wf_ref, bf_ref,
                 wb_ref, bb_ref, hf_out, hb_out,
                 hf_c, cf_c, hb_c, cb_c):
    @pl.when(pl.program_id(0) == 0)
    def _init():
        z = jnp.zeros((B, H), jnp.float32)
        hf_c[...] = z
        cf_c[...] = z
        hb_c[...] = z
        cb_c[...] = z

    h2, c2 = _lstm_step(ef_ref[0], hf_c[...], cf_c[...],
                        wf_ref[...], bf_ref[...])
    hf_out[0] = h2
    hf_c[...] = h2
    cf_c[...] = c2

    h2, c2 = _lstm_step(eb_ref[0], hb_c[...], cb_c[...],
                        wb_ref[...], bb_ref[...])
    hb_out[0] = h2
    hb_c[...] = h2
    cb_c[...] = c2


def _lstm_tc(emb_t, w_f, b_f, w_b, b_b):
    full2 = lambda t: (0, 0)
    full1 = lambda t: (0,)
    return pl.pallas_call(
        _lstm_kernel,
        grid=(L,),
        in_specs=[
            pl.BlockSpec((1, B, 2 * D), lambda t: (t, 0, 0)),
            pl.BlockSpec((1, B, 2 * D), lambda t: (L - 1 - t, 0, 0)),
            pl.BlockSpec((2 * H, 4 * H), full2),
            pl.BlockSpec((4 * H,), full1),
            pl.BlockSpec((2 * H, 4 * H), full2),
            pl.BlockSpec((4 * H,), full1),
        ],
        out_specs=[
            pl.BlockSpec((1, B, H), lambda t: (t, 0, 0)),
            pl.BlockSpec((1, B, H), lambda t: (L - 1 - t, 0, 0)),
        ],
        out_shape=[
            jax.ShapeDtypeStruct((L, B, H), jnp.float32),
            jax.ShapeDtypeStruct((L, B, H), jnp.float32),
        ],
        scratch_shapes=[pltpu.VMEM((B, H), jnp.float32)] * 4,
    )(emb_t, emb_t, w_f, b_f, w_b, b_b)


def _heads_kernel(e_ref, hf_ref, hb_ref,
                  c1w_ref, c1b_ref, c2w_ref, c2b_ref,
                  l1wf_ref, l1wb_ref, l1b_ref, l2w_ref, l2b_ref,
                  l3w_ref, l3b_ref,
                  cs1wf_ref, cs1wb_ref, cs1b_ref, cs2wt_ref, cs2b_ref,
                  f1wf_ref, f1wb_ref, f1b_ref, f2w_ref, f2b_ref,
                  lang_ref, cs_ref, form_ref, char_ref):
    dot = functools.partial(jnp.dot, preferred_element_type=jnp.float32)

    e = e_ref[...].astype(jnp.bfloat16)
    zp = jnp.zeros((2, _BB, 2 * D), jnp.bfloat16)
    ep = jnp.concatenate([zp, e, zp], axis=0)
    acc = None
    for k in range(3):
        xs = ep[1 + k:1 + k + L].reshape(L * _BB, 2 * D)
        t = dot(xs, c1w_ref[k])
        acc = t if acc is None else acc + t
    y1 = jax.nn.relu(acc + c1b_ref[...])
    y116 = y1.astype(jnp.bfloat16)
    zp1 = jnp.zeros((2, _BB, 128), jnp.bfloat16)
    y1p = jnp.concatenate([zp1, y116.reshape(L, _BB, 128), zp1], axis=0)
    acc = None
    for k in range(5):
        xs = y1p[k:k + L].reshape(L * _BB, 128)
        t = dot(xs, c2w_ref[k])
        acc = t if acc is None else acc + t
    y2 = jax.nn.relu(acc + c2b_ref[...])
    char_ref[...] = jnp.max(y2.reshape(L, _BB, 128), axis=0)

    hf = hf_ref[...]
    hb = hb_ref[...]
    gapf = jnp.mean(hf, axis=0)
    gapb = jnp.mean(hb, axis=0)
    z1 = jax.nn.relu(dot(gapf, l1wf_ref[...]) + dot(gapb, l1wb_ref[...])
                     + l1b_ref[...])
    z2 = jax.nn.relu(dot(z1, l2w_ref[...]) + l2b_ref[...])
    logits = dot(z2, l3w_ref[...]) + l3b_ref[...]
    lang_ref[...] = jax.nn.softmax(logits, axis=-1)

    fz = jax.nn.relu(dot(gapf, f1wf_ref[...]) + dot(gapb, f1wb_ref[...])
                     + f1b_ref[...])
    flog = dot(fz, f2w_ref[...]) + f2b_ref[...]
    form_ref[...] = jax.nn.softmax(flog, axis=-1)

    hff = hf.reshape(L * _BB, H).astype(jnp.bfloat16)
    hbf = hb.reshape(L * _BB, H).astype(jnp.bfloat16)
    csz = jax.nn.relu(dot(hff, cs1wf_ref[...]) + dot(hbf, cs1wb_ref[...])
                      + cs1b_ref[...])
    csv = jnp.sum(csz * cs2wt_ref[...], axis=1) + cs2b_ref[...]
    sig = jax.nn.sigmoid(csv).reshape(L, _BB)
    cs_ref[...] = sig.T.reshape(_BB, L, 1)


def _heads_tc(emb_t, hf, hb, c1w, c1b, c2w, c2b,
              l1wf, l1wb, l1b, l2w, l2b, l3w, l3b,
              cs1wf, cs1wb, cs1b, cs2wt, cs2b,
              f1wf, f1wb, f1b, f2w, f2b):
    def full(shape):
        n = len(shape)
        return pl.BlockSpec(shape, lambda i, _n=n: (0,) * _n)

    return pl.pallas_call(
        _heads_kernel,
        grid=(B // _BB,),
        in_specs=[
            pl.BlockSpec((L, _BB, 2 * D), lambda i: (0, i, 0)),
            pl.BlockSpec((L, _BB, H), lambda i: (0, i, 0)),
            pl.BlockSpec((L, _BB, H), lambda i: (0, i, 0)),
            full((3, 2 * D, 128)), full((128,)),
            full((5, 128, 128)), full((128,)),
            full((H, 256)), full((H, 256)), full((256,)),
            full((256, 128)), full((128,)),
            full((128, NUM_LANG)), full((NUM_LANG,)),
            full((H, 64)), full((H, 64)), full((64,)),
            full((1, 64)), full((1,)),
            full((H, 64)), full((H, 64)), full((64,)),
            full((64, 3)), full((3,)),
        ],
        out_specs=[
            pl.BlockSpec((_BB, NUM_LANG), lambda i: (i, 0)),
            pl.BlockSpec((_BB, L, 1), lambda i: (i, 0, 0)),
            pl.BlockSpec((_BB, 3), lambda i: (i, 0)),
            pl.BlockSpec((_BB, 128), lambda i: (i, 0)),
        ],
        out_shape=[
            jax.ShapeDtypeStruct((B, NUM_LANG), jnp.float32),
            jax.ShapeDtypeStruct((B, L, 1), jnp.float32),
            jax.ShapeDtypeStruct((B, 3), jnp.float32),
            jax.ShapeDtypeStruct((B, 128), jnp.float32),
        ],
    )(emb_t, hf, hb, c1w, c1b, c2w, c2b,
      l1wf, l1wb, l1b, l2w, l2b, l3w, l3b,
      cs1wf, cs1wb, cs1b, cs2wt, cs2b,
      f1wf, f1wb, f1b, f2w, f2b)


def kernel(token_ids, table, c1w, c1b, c2w, c2b, wx_f, wh_f, b_f,
           wx_b, wh_b, b_b, l1w, l1b, l2w, l2b, l3w, l3b,
           cs1w, cs1b, cs2w, cs2b, f1w, f1b, f2w, f2b):
    tok128 = _pad_tok_tc(token_ids.astype(jnp.int32))
    table_w = _widen_tc(table.T)
    emb_t = _sc_gather(table_w, tok128).reshape(L, B, 2 * D)
    zpad = jnp.zeros((D, 4 * H), jnp.float32)
    w_f = jnp.concatenate([wx_f, zpad, wh_f], axis=0)
    w_b = jnp.concatenate([wx_b, zpad, wh_b], axis=0)
    hf, hb = _lstm_tc(emb_t, w_f, b_f, w_b, b_b)
    c1wp = jnp.pad(c1w, ((0, 0), (0, D), (0, 0))).astype(jnp.bfloat16)
    lang, code_switch, form, char = _heads_tc(
        emb_t, hf, hb, c1wp, c1b, c2w.astype(jnp.bfloat16), c2b,
        l1w[:H], l1w[H:], l1b, l2w, l2b, l3w, l3b,
        cs1w[:H].astype(jnp.bfloat16), cs1w[H:].astype(jnp.bfloat16),
        cs1b, cs2w.T, cs2b,
        f1w[:H], f1w[H:], f1b, f2w, f2b)
    return lang, code_switch, form, char
